# baseline (speedup 1.0000x reference)
import functools

import jax
import jax.numpy as jnp
from jax import lax
from jax.experimental import pallas as pl
from jax.experimental.pallas import tpu as pltpu

_GC = 40
_GP = 192


def _sppcspc_kernel(H, W,
                    x_ref,
                    w01_ref, b01_ref, wb_ref, bsm_ref, w11_ref, b11_ref,
                    o_ref,
                    s_conv2, s_x02, s_x12, s_a52, s_a92, s_a132):
    HW = H * W
    f32 = jnp.float32
    bf16 = jnp.bfloat16

    row = lax.broadcasted_iota(jnp.int32, (HW, 1), 0)
    ww = jnp.bitwise_and(row, W - 1)

    def w_valid(dw):
        return (ww + dw >= 0) & (ww + dw < W)

    ninf = jnp.array(-jnp.inf, bf16)

    def silu_f(y):
        return y * jax.nn.sigmoid(y)

    def silu_q(y):
        yb = y.astype(bf16)
        return yb * lax.logistic(yb)

    def wshift(val, dw):
        return pltpu.roll(val, (-dw) % HW, 0)

    def cbs3x3(center, s_conv, base, b_r):
        rc = jnp.concatenate(
            [s_conv[pl.ds(_GC - W, HW), :], center,
             s_conv[pl.ds(_GC + W, HW), :]], axis=1)

        def col(kw):
            wcol = jnp.concatenate(
                [wb_ref[base + kh * 3 + kw] for kh in range(3)], axis=0)
            return jnp.dot(rc, wcol, preferred_element_type=f32)

        acc = (col(1)
               + jnp.where(w_valid(-1), wshift(col(0), -1), 0.0)
               + jnp.where(w_valid(1), wshift(col(2), 1), 0.0))
        return silu_q(acc + b_r[...])

    def wmax(center, pieces):
        r = center
        for dw, val in pieces:
            r = jnp.maximum(r, jnp.where(w_valid(dw), wshift(val, dw), ninf))
        return r

    def hmax(center, src_ref, radius):
        out = center
        for dh in range(-radius, radius + 1):
            if dh != 0:
                out = jnp.maximum(out, src_ref[pl.ds(_GP + dh * W, HW), :])
        return out

    def one(e):
        s_conv = s_conv2.at[e]
        s_x0 = s_x02.at[e]
        s_x1 = s_x12.at[e]
        s_a5 = s_a52.at[e]
        s_a9 = s_a92.at[e]
        s_a13 = s_a132.at[e]

        s_conv[:_GC, :] = jnp.full((_GC, s_conv2.shape[2]), 0.0, bf16)
        s_conv[_GC + HW:, :] = jnp.full((_GC, s_conv2.shape[2]), 0.0, bf16)
        for sc in (s_a5, s_a9, s_a13):
            sc[:_GP, :] = jnp.full((_GP, s_a52.shape[2]), -jnp.inf, bf16)
            sc[_GP + HW:, :] = jnp.full((_GP, s_a52.shape[2]), -jnp.inf, bf16)

        xb = x_ref[e].astype(bf16)

        h01 = lax.dot_general(xb, w01_ref[...], (((0,), (0,)), ((), ())),
                              preferred_element_type=f32) + b01_ref[...]
        s01 = silu_q(h01)
        half = s01.shape[1] // 2
        s_x0[...] = s01[:, :half]
        t1 = s01[:, half:]
        s_conv[pl.ds(_GC, HW), :] = t1

        t2 = cbs3x3(t1, s_conv, 0, bsm_ref[0:1, :])

        x1 = silu_q(jnp.dot(t2, wb_ref[22], preferred_element_type=f32)
                    + bsm_ref[1:2, :])
        s_x1[...] = x1

        a5 = wmax(x1, [(-2, x1), (-1, x1), (1, x1), (2, x1)])
        s_a5[pl.ds(_GP, HW), :] = a5
        a9 = wmax(a5, [(-2, a5), (2, a5)])
        s_a9[pl.ds(_GP, HW), :] = a9
        a13 = wmax(a9, [(-4, a5), (4, a5)])
        s_a13[pl.ds(_GP, HW), :] = a13

        p5 = hmax(a5, s_a5, 2)
        h9 = hmax(a9, s_a9, 2)
        s_a5[pl.ds(_GP, HW), :] = h9
        p9 = jnp.maximum(h9, jnp.maximum(
            s_a5[pl.ds(_GP - 2 * W, HW), :],
            s_a5[pl.ds(_GP + 2 * W, HW), :]))
        h13 = hmax(a13, s_a13, 2)
        s_a9[pl.ds(_GP, HW), :] = h13
        p13 = jnp.maximum(h13, jnp.maximum(
            s_a9[pl.ds(_GP - 4 * W, HW), :],
            s_a9[pl.ds(_GP + 4 * W, HW), :]))

        y = (jnp.dot(s_x1[...], wb_ref[18], preferred_element_type=f32)
             + jnp.dot(p5, wb_ref[19], preferred_element_type=f32)
             + jnp.dot(p9, wb_ref[20], preferred_element_type=f32)
             + jnp.dot(p13, wb_ref[21], preferred_element_type=f32)
             + bsm_ref[2:3, :])
        y = silu_q(y)
        s_conv[pl.ds(_GC, HW), :] = y

        y2 = cbs3x3(y, s_conv, 9, bsm_ref[3:4, :])

        out = (lax.dot_general(w11_ref[0], y2, (((1,), (1,)), ((), ())),
                               preferred_element_type=f32)
               + lax.dot_general(w11_ref[1], s_x0[...], (((1,), (1,)), ((), ())),
                                 preferred_element_type=f32)
               + b11_ref[...])
        o_ref[e] = silu_f(out).astype(o_ref.dtype)

    one(0)
    one(1)


@jax.jit
def _sppcspc_forward(x_nchw, *weights):
    N, C, H, W = x_nchw.shape
    HW = H * W
    x3 = x_nchw.reshape(N, C, HW)
    n_out = weights[-2].shape[1]
    half = weights[-2].shape[2]

    def const_spec(a):
        nd = a.ndim
        return pl.BlockSpec(a.shape, lambda n: (0,) * nd)

    kern = functools.partial(_sppcspc_kernel, H, W)
    out3 = pl.pallas_call(
        kern,
        out_shape=jax.ShapeDtypeStruct((N, n_out, HW), jnp.float32),
        grid=(N // 2,),
        in_specs=[pl.BlockSpec((2, C, HW), lambda n: (n, 0, 0))]
                 + [const_spec(w) for w in weights],
        out_specs=pl.BlockSpec((2, n_out, HW), lambda n: (n, 0, 0)),
        scratch_shapes=[
            pltpu.VMEM((2, HW + 2 * _GC, half), jnp.bfloat16),
            pltpu.VMEM((2, HW, half), jnp.bfloat16),
            pltpu.VMEM((2, HW, half), jnp.bfloat16),
            pltpu.VMEM((2, HW + 2 * _GP, half), jnp.bfloat16),
            pltpu.VMEM((2, HW + 2 * _GP, half), jnp.bfloat16),
            pltpu.VMEM((2, HW + 2 * _GP, half), jnp.bfloat16),
        ],
        compiler_params=pltpu.CompilerParams(dimension_semantics=("parallel",)),
    )(x3, *weights)
    return out3.reshape(N, n_out, H, W)


def kernel(x, w0, b0, w1, b1, w2, b2, w3, b3, w8, b8, w9, b9, w11, b11):
    bf = jnp.bfloat16
    f32 = jnp.float32
    w01 = jnp.concatenate([w0, w1], axis=0).T.astype(bf)
    b01 = jnp.concatenate([b0, b1], axis=0).reshape(1, -1).astype(f32)
    wb = jnp.concatenate([w2, w9, w8, w3[None]], axis=0)
    wb = jnp.transpose(wb, (0, 2, 1)).astype(bf)
    bsm = jnp.concatenate([b2, b3, b8, b9], axis=1).T.astype(f32)
    ws = (w01, b01, wb, bsm, w11.astype(bf), b11.astype(f32))
    return _sppcspc_forward(x, *ws)


# speedup vs baseline: 1.0034x; 1.0034x over previous
import functools

import jax
import jax.numpy as jnp
from jax import lax
from jax.experimental import pallas as pl
from jax.experimental.pallas import tpu as pltpu

_GC = 40
_GP = 192


def _sppcspc_kernel(H, W,
                    x_ref,
                    w01_ref, b01_ref, wb_ref, bsm_ref, w11_ref, b11_ref,
                    o_ref,
                    s_conv2, s_x02, s_a52, s_a92, s_a132):
    HW = H * W
    f32 = jnp.float32
    bf16 = jnp.bfloat16

    row = lax.broadcasted_iota(jnp.int32, (HW, 1), 0)
    ww = jnp.bitwise_and(row, W - 1)

    def w_valid(dw):
        return (ww + dw >= 0) & (ww + dw < W)

    ninf = jnp.array(-jnp.inf, bf16)

    def silu_f(y):
        return y * jax.nn.sigmoid(y)

    def silu_q(y):
        yb = y.astype(bf16)
        return yb * lax.logistic(yb)

    def wshift(val, dw):
        return pltpu.roll(val, (-dw) % HW, 0)

    def cbs3x3(center, s_conv, base, b_r):
        rc = jnp.concatenate(
            [s_conv[pl.ds(_GC - W, HW), :], center,
             s_conv[pl.ds(_GC + W, HW), :]], axis=1)

        def col(kw):
            wcol = jnp.concatenate(
                [wb_ref[base + kh * 3 + kw] for kh in range(3)], axis=0)
            return jnp.dot(rc, wcol, preferred_element_type=f32)

        acc = (col(1)
               + jnp.where(w_valid(-1), wshift(col(0), -1), 0.0)
               + jnp.where(w_valid(1), wshift(col(2), 1), 0.0))
        return silu_q(acc + b_r[...])

    def wmax(center, pieces):
        r = center
        for dw, val in pieces:
            r = jnp.maximum(r, jnp.where(w_valid(dw), wshift(val, dw), ninf))
        return r

    def hmax(center, src_ref, radius):
        out = center
        for dh in range(-radius, radius + 1):
            if dh != 0:
                out = jnp.maximum(out, src_ref[pl.ds(_GP + dh * W, HW), :])
        return out

    def one(e):
        s_conv = s_conv2.at[e]
        s_x0 = s_x02.at[e]
        s_a5 = s_a52.at[e]
        s_a9 = s_a92.at[e]
        s_a13 = s_a132.at[e]

        s_conv[:_GC, :] = jnp.full((_GC, s_conv2.shape[2]), 0.0, bf16)
        s_conv[_GC + HW:, :] = jnp.full((_GC, s_conv2.shape[2]), 0.0, bf16)
        for sc in (s_a5, s_a9, s_a13):
            sc[:_GP, :] = jnp.full((_GP, s_a52.shape[2]), -jnp.inf, bf16)
            sc[_GP + HW:, :] = jnp.full((_GP, s_a52.shape[2]), -jnp.inf, bf16)

        xb = x_ref[e].astype(bf16)

        h01 = lax.dot_general(xb, w01_ref[...], (((0,), (0,)), ((), ())),
                              preferred_element_type=f32) + b01_ref[...]
        s01 = silu_q(h01)
        half = s01.shape[1] // 2
        s_x0[...] = s01[:, :half]
        t1 = s01[:, half:]
        s_conv[pl.ds(_GC, HW), :] = t1

        t2 = cbs3x3(t1, s_conv, 0, bsm_ref[0:1, :])

        x1 = silu_q(jnp.dot(t2, wb_ref[22], preferred_element_type=f32)
                    + bsm_ref[1:2, :])

        a5 = wmax(x1, [(-2, x1), (-1, x1), (1, x1), (2, x1)])
        s_a5[pl.ds(_GP, HW), :] = a5
        a9 = wmax(a5, [(-2, a5), (2, a5)])
        s_a9[pl.ds(_GP, HW), :] = a9
        a13 = wmax(a9, [(-4, a5), (4, a5)])
        s_a13[pl.ds(_GP, HW), :] = a13

        p5 = hmax(a5, s_a5, 2)
        h9 = hmax(a9, s_a9, 2)
        s_a5[pl.ds(_GP, HW), :] = h9
        p9 = jnp.maximum(h9, jnp.maximum(
            s_a5[pl.ds(_GP - 2 * W, HW), :],
            s_a5[pl.ds(_GP + 2 * W, HW), :]))
        h13 = hmax(a13, s_a13, 2)
        s_a9[pl.ds(_GP, HW), :] = h13
        p13 = jnp.maximum(h13, jnp.maximum(
            s_a9[pl.ds(_GP - 4 * W, HW), :],
            s_a9[pl.ds(_GP + 4 * W, HW), :]))

        y = (jnp.dot(x1, wb_ref[18], preferred_element_type=f32)
             + jnp.dot(p5, wb_ref[19], preferred_element_type=f32)
             + jnp.dot(p9, wb_ref[20], preferred_element_type=f32)
             + jnp.dot(p13, wb_ref[21], preferred_element_type=f32)
             + bsm_ref[2:3, :])
        y = silu_q(y)
        s_conv[pl.ds(_GC, HW), :] = y

        y2 = cbs3x3(y, s_conv, 9, bsm_ref[3:4, :])

        out = (lax.dot_general(w11_ref[0], y2, (((1,), (1,)), ((), ())),
                               preferred_element_type=f32)
               + lax.dot_general(w11_ref[1], s_x0[...], (((1,), (1,)), ((), ())),
                                 preferred_element_type=f32)
               + b11_ref[...])
        o_ref[e] = silu_f(out).astype(o_ref.dtype)

    one(0)
    one(1)


@jax.jit
def _sppcspc_forward(x_nchw, *weights):
    N, C, H, W = x_nchw.shape
    HW = H * W
    x3 = x_nchw.reshape(N, C, HW)
    n_out = weights[-2].shape[1]
    half = weights[-2].shape[2]

    def const_spec(a):
        nd = a.ndim
        return pl.BlockSpec(a.shape, lambda n: (0,) * nd)

    kern = functools.partial(_sppcspc_kernel, H, W)
    out3 = pl.pallas_call(
        kern,
        out_shape=jax.ShapeDtypeStruct((N, n_out, HW), jnp.float32),
        grid=(N // 2,),
        in_specs=[pl.BlockSpec((2, C, HW), lambda n: (n, 0, 0))]
                 + [const_spec(w) for w in weights],
        out_specs=pl.BlockSpec((2, n_out, HW), lambda n: (n, 0, 0)),
        scratch_shapes=[
            pltpu.VMEM((2, HW + 2 * _GC, half), jnp.bfloat16),
            pltpu.VMEM((2, HW, half), jnp.bfloat16),
            pltpu.VMEM((2, HW + 2 * _GP, half), jnp.bfloat16),
            pltpu.VMEM((2, HW + 2 * _GP, half), jnp.bfloat16),
            pltpu.VMEM((2, HW + 2 * _GP, half), jnp.bfloat16),
        ],
        compiler_params=pltpu.CompilerParams(dimension_semantics=("parallel",)),
    )(x3, *weights)
    return out3.reshape(N, n_out, H, W)


def kernel(x, w0, b0, w1, b1, w2, b2, w3, b3, w8, b8, w9, b9, w11, b11):
    bf = jnp.bfloat16
    f32 = jnp.float32
    w01 = jnp.concatenate([w0, w1], axis=0).T.astype(bf)
    b01 = jnp.concatenate([b0, b1], axis=0).reshape(1, -1).astype(f32)
    wb = jnp.concatenate([w2, w9, w8, w3[None]], axis=0)
    wb = jnp.transpose(wb, (0, 2, 1)).astype(bf)
    bsm = jnp.concatenate([b2, b3, b8, b9], axis=1).T.astype(f32)
    ws = (w01, b01, wb, bsm, w11.astype(bf), b11.astype(f32))
    return _sppcspc_forward(x, *ws)
